# baseline (speedup 1.0000x reference)
import functools

import jax
import jax.numpy as jnp
from jax import lax
from jax.experimental import pallas as pl
from jax.experimental.pallas import tpu as pltpu
from jax.experimental.pallas import tpu_sc as plsc

_MIN_NORM = 1e-15
_EPS = 4e-3

_N = 10000
_D = 128
_R_SC = 1600
_BM = 400
_NW = 32
_GROUPS = _N // 16


def _artanh(v):
    v = jnp.clip(v, -1.0 + 1e-7, 1.0 - 1e-7)
    return 0.5 * (jnp.log1p(v) - jnp.log1p(-v))


def _tangent_body(x_ref, o_ref):
    x = x_ref[...]
    n = jnp.sqrt(jnp.sum(x * x, axis=-1, keepdims=True))
    n = jnp.maximum(n, _MIN_NORM)
    o_ref[pl.ds(0, _N), :] = x / n * _artanh(n)
    o_ref[pl.ds(_N, 16), :] = jnp.zeros((16, _D), jnp.float32)


def _exp_proj(acc):
    n = jnp.maximum(jnp.sqrt(jnp.sum(acc * acc, axis=-1, keepdims=True)),
                    _MIN_NORM)
    y = jnp.tanh(n) * acc / n
    yn = jnp.maximum(jnp.sqrt(jnp.sum(y * y, axis=-1, keepdims=True)),
                     _MIN_NORM)
    maxnorm = 1.0 - _EPS
    return jnp.where(yn > maxnorm, y / yn * maxnorm, y)


def _agg_body(xt_ref, adj_ref, o_ref):
    acc = jnp.dot(adj_ref[...], xt_ref[pl.ds(0, _N), :],
                  preferred_element_type=jnp.float32)
    o_ref[...] = _exp_proj(acc)


def _epi_body(sup_ref, o_ref):
    o_ref[...] = _exp_proj(sup_ref[...])


def _sc_agg_body(xt_hbm, adj_hbm, out_hbm,
                 rowbuf, idx_buf, rows16, outbuf, sem_row, sem_g):
    rpw = _R_SC // _NW
    wid = lax.axis_index("s") * 2 + lax.axis_index("c")
    w_base = wid * rpw

    def row_body(r, _):
        pltpu.sync_copy(adj_hbm.at[pl.ds((w_base + r) * _N, _N)], rowbuf)

        def scan_body(j, cnt):
            v = rowbuf[pl.ds(16 * j, 16)]
            m = v != 0.0
            c = jnp.sum(m.astype(jnp.int32))

            @pl.when(c > 0)
            def _():
                lanes = lax.iota(jnp.int32, 16) + 16 * j
                plsc.store_compressed(idx_buf.at[pl.ds(cnt, 16)], lanes,
                                      mask=m)

            return cnt + c

        cnt = lax.fori_loop(0, _GROUPS, scan_body, jnp.int32(0))
        num_g = (cnt + 15) // 16
        idx_buf[pl.ds(cnt, 16)] = jnp.full((16,), _N, jnp.int32)

        def g_body(g, acc):
            iv = idx_buf[pl.ds(16 * g, 16)]
            pltpu.async_copy(xt_hbm.at[iv], rows16, sem_g).wait()
            new = []
            for c8 in range(8):
                a = acc[c8]
                for t in range(16):
                    a = a + rows16[t, pl.ds(16 * c8, 16)]
                new.append(a)
            return tuple(new)

        acc0 = tuple(jnp.zeros((16,), jnp.float32) for _ in range(8))
        acc = lax.fori_loop(0, num_g, g_body, acc0)
        wv = 1.0 / jnp.maximum(jnp.full((16,), cnt.astype(jnp.float32)), 1.0)
        for c8 in range(8):
            outbuf[pl.ds(r * _D + 16 * c8, 16)] = acc[c8] * wv
        return 0

    lax.fori_loop(0, rpw, row_body, 0)
    pltpu.sync_copy(outbuf, out_hbm.at[pl.ds(w_base * _D, rpw * _D)])


def _sc_agg(xt, adj):
    rpw = _R_SC // _NW
    mesh = plsc.VectorSubcoreMesh(core_axis_name="c", subcore_axis_name="s")
    f = pl.kernel(
        _sc_agg_body,
        mesh=mesh,
        compiler_params=pltpu.CompilerParams(needs_layout_passes=False),
        out_type=jax.ShapeDtypeStruct((_R_SC * _D,), jnp.float32),
        scratch_types=[
            pltpu.VMEM((_N,), jnp.float32),
            pltpu.VMEM((_N + 16,), jnp.int32),
            pltpu.VMEM((16, _D), jnp.float32),
            pltpu.VMEM((rpw * _D,), jnp.float32),
            pltpu.SemaphoreType.DMA,
            pltpu.SemaphoreType.DMA,
        ],
    )
    return jnp.reshape(f(xt, jnp.reshape(adj, (-1,))), (_R_SC, _D))


def kernel(x, adj):
    n_nodes, d = x.shape
    xt = pl.pallas_call(
        _tangent_body,
        grid=(1,),
        in_specs=[pl.BlockSpec((n_nodes, d), lambda i: (0, 0))],
        out_specs=pl.BlockSpec((n_nodes + 16, d), lambda i: (0, 0)),
        out_shape=jax.ShapeDtypeStruct((n_nodes + 16, d), jnp.float32),
    )(x)

    sc_sup = _sc_agg(xt, adj)
    sc_out = pl.pallas_call(
        _epi_body,
        grid=(_R_SC // _BM,),
        in_specs=[pl.BlockSpec((_BM, d), lambda i: (i, 0))],
        out_specs=pl.BlockSpec((_BM, d), lambda i: (i, 0)),
        out_shape=jax.ShapeDtypeStruct((_R_SC, d), jnp.float32),
    )(sc_sup)

    nblk = _R_SC // _BM
    tc_rows = n_nodes - _R_SC
    tc_out = pl.pallas_call(
        _agg_body,
        grid=(tc_rows // _BM,),
        in_specs=[
            pl.BlockSpec((n_nodes + 16, d), lambda i: (0, 0)),
            pl.BlockSpec((_BM, n_nodes), lambda i: (i + nblk, 0)),
        ],
        out_specs=pl.BlockSpec((_BM, d), lambda i: (i, 0)),
        out_shape=jax.ShapeDtypeStruct((tc_rows, d), jnp.float32),
    )(xt, adj)

    return jnp.concatenate([sc_out, tc_out], axis=0)
